# baseline (speedup 1.0000x reference)
import functools

import jax
import jax.numpy as jnp
from jax import lax
from jax.experimental import pallas as pl
from jax.experimental.pallas import tpu as pltpu
from jax.experimental.pallas import tpu_sc as plsc

HIDDEN = 768
FFN = 1024
NUM_EXPERTS = 8
TOPK = 2
T = 2048
LANES = 128
NEG = -1e30
BM = 128
S = TOPK * T + NUM_EXPERTS * BM
NBLK = S // BM
L = 16


def _routing_body(x_ref, gw_ref, d0_ref, d1_ref, w0_ref, w1_ref,
                  bexp_ref, bval_ref):
    x = x_ref[...]
    gw = gw_ref[...]
    logits = lax.dot_general(
        x, gw, (((1,), (1,)), ((), ())), preferred_element_type=jnp.float32
    )
    lane = lax.broadcasted_iota(jnp.int32, (T, LANES), 1)
    valid = lane < NUM_EXPERTS
    logits = jnp.where(valid, logits, NEG)
    m = jnp.max(logits, axis=1, keepdims=True)
    p = jnp.where(valid, jnp.exp(logits - m), 0.0)
    m1 = jnp.max(p, axis=1, keepdims=True)
    a1 = jnp.min(jnp.where(p == m1, lane, LANES), axis=1, keepdims=True)
    oh1 = (lane == a1)
    p2 = jnp.where(oh1, 0.0, p)
    m2 = jnp.max(p2, axis=1, keepdims=True)
    a2 = jnp.min(jnp.where(p2 == m2, lane, LANES), axis=1, keepdims=True)
    oh2 = (lane == a2)
    s = m1 + m2
    w0_ref[...] = jnp.broadcast_to(m1 / s, (T, L))
    w1_ref[...] = jnp.broadcast_to(m2 / s, (T, L))

    c = oh1.astype(jnp.float32) + oh2.astype(jnp.float32)
    cum = c
    d = 1
    while d < T:
        cum = cum + jnp.concatenate(
            [jnp.zeros((d, LANES), jnp.float32), cum[:T - d]], axis=0)
        d *= 2
    xexcl = (cum - c).astype(jnp.int32)
    counts = cum[T - 1:T, :].astype(jnp.int32)
    padded = ((counts + BM - 1) // BM) * BM
    offi = padded
    d = 1
    while d < LANES:
        offi = offi + jnp.concatenate(
            [jnp.zeros((1, d), jnp.int32), offi[:, :LANES - d]], axis=1)
        d *= 2
    off = offi - padded
    oh1i = oh1.astype(jnp.int32)
    oh2i = oh2.astype(jnp.int32)
    d0_ref[...] = jnp.sum((off + xexcl) * oh1i, axis=1, keepdims=True)
    d1_ref[...] = jnp.sum((off + xexcl + oh1i) * oh2i, axis=1, keepdims=True)

    bs = lane[:1, :] * BM
    be = jnp.zeros((1, LANES), jnp.int32)
    end_sel = jnp.zeros((1, LANES), jnp.int32)
    for e in range(NUM_EXPERTS):
        sel = (lane[:1, :] == e).astype(jnp.int32)
        off_e = jnp.sum(off * sel, axis=1, keepdims=True)
        end_e = jnp.sum((off + counts) * sel, axis=1, keepdims=True)
        be = be + (off_e <= bs).astype(jnp.int32)
    be = jnp.maximum(be - 1, 0)
    for e in range(NUM_EXPERTS):
        sel = (lane[:1, :] == e).astype(jnp.int32)
        end_e = jnp.sum((off + counts) * sel, axis=1, keepdims=True)
        end_sel = end_sel + (be == e).astype(jnp.int32) * end_e
    bexp_ref[...] = be
    bval_ref[...] = (bs < end_sel).astype(jnp.int32)


def _routing(x, gw_pad):
    return pl.pallas_call(
        _routing_body,
        out_shape=[
            jax.ShapeDtypeStruct((T, 1), jnp.int32),
            jax.ShapeDtypeStruct((T, 1), jnp.int32),
            jax.ShapeDtypeStruct((T, L), jnp.float32),
            jax.ShapeDtypeStruct((T, L), jnp.float32),
            jax.ShapeDtypeStruct((1, LANES), jnp.int32),
            jax.ShapeDtypeStruct((1, LANES), jnp.int32),
        ],
    )(x, gw_pad)


def _gmm_body(meta_ref, xs_ref, w13_ref, w2_ref, ys_ref):
    b = pl.program_id(0)

    @pl.when(meta_ref[NBLK + b] == 1)
    def _():
        xb = xs_ref[...]
        h = lax.dot_general(
            xb, w13_ref[0], (((1,), (1,)), ((), ())),
            preferred_element_type=jnp.float32,
        )
        h1 = h[:, :FFN]
        h3 = h[:, FFN:]
        inter = h1 * (1.0 / (1.0 + jnp.exp(-h1))) * h3
        ys_ref[...] = lax.dot_general(
            inter, w2_ref[0], (((1,), (1,)), ((), ())),
            preferred_element_type=jnp.float32,
        )


def _gmm(meta, xs, w13, w2):
    return pl.pallas_call(
        _gmm_body,
        grid_spec=pltpu.PrefetchScalarGridSpec(
            num_scalar_prefetch=1,
            grid=(NBLK,),
            in_specs=[
                pl.BlockSpec((BM, HIDDEN), lambda b, m: (b, 0)),
                pl.BlockSpec((1, 2 * FFN, HIDDEN), lambda b, m: (m[b], 0, 0)),
                pl.BlockSpec((1, HIDDEN, FFN), lambda b, m: (m[b], 0, 0)),
            ],
            out_specs=pl.BlockSpec((BM, HIDDEN), lambda b, m: (b, 0)),
        ),
        out_shape=jax.ShapeDtypeStruct((S, HIDDEN), jnp.float32),
    )(meta, xs, w13, w2)


def _dispatch_body(tpw, nc, x_hbm, d0_hbm, d1_hbm, xs_hbm,
                   xrows_v, d0_v, d1_v, sem):
    wid = lax.axis_index("s") * nc + lax.axis_index("c")
    base = wid * tpw
    pltpu.sync_copy(x_hbm.at[pl.ds(base, tpw)], xrows_v)
    pltpu.sync_copy(d0_hbm.at[pl.ds(base, tpw)], d0_v)
    pltpu.sync_copy(d1_hbm.at[pl.ds(base, tpw)], d1_v)
    pltpu.async_copy(xrows_v, xs_hbm.at[d0_v], sem).wait()
    pltpu.async_copy(xrows_v, xs_hbm.at[d1_v], sem).wait()


def _combine_body(tpw, nc, ys_hbm, d0_hbm, d1_hbm, w0_hbm, w1_hbm, out_hbm,
                  ra_v, rb_v, d0_v, d1_v, w0_v, w1_v, sem):
    wid = lax.axis_index("s") * nc + lax.axis_index("c")
    base = wid * tpw
    pltpu.sync_copy(d0_hbm.at[pl.ds(base, tpw)], d0_v)
    pltpu.sync_copy(d1_hbm.at[pl.ds(base, tpw)], d1_v)
    pltpu.sync_copy(w0_hbm.at[pl.ds(base, tpw)], w0_v)
    pltpu.sync_copy(w1_hbm.at[pl.ds(base, tpw)], w1_v)
    pltpu.async_copy(ys_hbm.at[d0_v], ra_v, sem).wait()
    pltpu.async_copy(ys_hbm.at[d1_v], rb_v, sem).wait()

    def row(j, _):
        w0b = w0_v[j, :]
        w1b = w1_v[j, :]
        for cch in range(HIDDEN // L):
            sl = pl.ds(cch * L, L)
            ra_v[j, sl] = w0b * ra_v[j, sl] + w1b * rb_v[j, sl]
        return 0

    lax.fori_loop(0, tpw, row, 0)
    pltpu.sync_copy(ra_v, out_hbm.at[pl.ds(base, tpw)])


def kernel(x, gate_w, w13, w2):
    gw_pad = jnp.zeros((LANES, HIDDEN), jnp.float32).at[:NUM_EXPERTS].set(gate_w)
    d0, d1, w0, w1, bexp, bval = _routing(x, gw_pad)
    d0 = d0.reshape(T)
    d1 = d1.reshape(T)
    meta = jnp.concatenate([bexp[0, :NBLK], bval[0, :NBLK]])

    info = plsc.get_sparse_core_info()
    nc, ns = info.num_cores, info.num_subcores
    nw = nc * ns
    tpw = T // nw
    mesh = plsc.VectorSubcoreMesh(core_axis_name="c", subcore_axis_name="s",
                                  num_cores=nc, num_subcores=ns)

    dispatch = functools.partial(
        pl.kernel,
        mesh=mesh,
        out_type=jax.ShapeDtypeStruct((S, HIDDEN), jnp.float32),
        scratch_types=[
            pltpu.VMEM((tpw, HIDDEN), jnp.float32),
            pltpu.VMEM((tpw,), jnp.int32),
            pltpu.VMEM((tpw,), jnp.int32),
            pltpu.SemaphoreType.DMA,
        ],
    )(functools.partial(_dispatch_body, tpw, nc))
    xs = dispatch(x, d0, d1)

    ys = _gmm(meta, xs, w13, w2)

    combine = functools.partial(
        pl.kernel,
        mesh=mesh,
        out_type=jax.ShapeDtypeStruct((T, HIDDEN), jnp.float32),
        scratch_types=[
            pltpu.VMEM((tpw, HIDDEN), jnp.float32),
            pltpu.VMEM((tpw, HIDDEN), jnp.float32),
            pltpu.VMEM((tpw,), jnp.int32),
            pltpu.VMEM((tpw,), jnp.int32),
            pltpu.VMEM((tpw, L), jnp.float32),
            pltpu.VMEM((tpw, L), jnp.float32),
            pltpu.SemaphoreType.DMA,
        ],
    )(functools.partial(_combine_body, tpw, nc))
    return combine(ys, d0, d1, w0, w1)
